# baseline (speedup 1.0000x reference)
import jax
import jax.numpy as jnp
from jax import lax
from jax.experimental import pallas as pl
from jax.experimental.pallas import tpu as pltpu

_SF = 2
_D = 28 * _SF
_DX = 172 * _SF
_DOUT = 3 * _D + _DX
_K = 64
_BB = 256


def _fused_body(xt_ref, w_ref, out_ref):
    xt = xt_ref[...]
    _, l, bb = xt.shape
    t1 = xt[0:1].astype(jnp.int32)
    t2 = xt[1:2].astype(jnp.int32) + 20
    t3 = xt[2:3].astype(jnp.int32) + 25

    c = lax.broadcasted_iota(jnp.int32, (_K, l, bb), 0)
    ones = (c == t1) | (c == t2) | (c == t3) | (c == 49)
    feats = ones.astype(jnp.float32)
    feats += jnp.where(c == 46, xt[3:4], 0.0)
    feats += jnp.where(c == 47, xt[4:5], 0.0)
    feats += jnp.where(c == 48, xt[5:6], 0.0)

    y = lax.dot_general(feats, w_ref[...], (((0,), (0,)), ((), ())),
                        preferred_element_type=jnp.float32)
    out_ref[...] = jnp.maximum(y, w_ref[50:51, :][None])


def _pack_weights(amino_table, element_table, position_table, W_xyz, b_xyz):
    w = jnp.zeros((_K, _DOUT), dtype=jnp.float32)
    w = w.at[0:20, 0:_D].set(amino_table)
    w = w.at[20:25, _D:2 * _D].set(element_table)
    w = w.at[25:46, 2 * _D:3 * _D].set(position_table)
    w = w.at[46:49, 3 * _D:].set(W_xyz)
    w = w.at[49, 3 * _D:].set(b_xyz)
    w = w.at[50, 0:3 * _D].set(jnp.finfo(jnp.float32).min)
    return w


def kernel(x, amino_table, element_table, position_table, W_xyz, b_xyz):
    B, L, _ = x.shape
    w = _pack_weights(amino_table, element_table, position_table, W_xyz,
                      b_xyz)
    xt = jnp.transpose(x, (2, 1, 0))
    out_t = pl.pallas_call(
        _fused_body,
        grid=(B // _BB, 2),
        in_specs=[
            pl.BlockSpec((6, L, _BB), lambda i, j: (0, 0, i)),
            pl.BlockSpec((_K, _DOUT // 2), lambda i, j: (0, j)),
        ],
        out_specs=pl.BlockSpec((L, _BB, _DOUT // 2), lambda i, j: (0, i, j)),
        out_shape=jax.ShapeDtypeStruct((L, B, _DOUT), jnp.float32),
        compiler_params=pltpu.CompilerParams(
            dimension_semantics=("parallel", "arbitrary"),
        ),
    )(xt, w)
    return jnp.transpose(out_t, (1, 0, 2))


# speedup vs baseline: 1.0486x; 1.0486x over previous
import jax
import jax.numpy as jnp
from jax import lax
from jax.experimental import pallas as pl
from jax.experimental.pallas import tpu as pltpu

_SF = 2
_D = 28 * _SF
_DX = 172 * _SF
_DOUT = 3 * _D + _DX
_K = 64
_BB = 128


def _fused_body(xt_ref, w_ref, out_ref):
    xt = xt_ref[...]
    _, l, bb = xt.shape
    t1 = xt[0:1].astype(jnp.int32)
    t2 = xt[1:2].astype(jnp.int32) + 20
    t3 = xt[2:3].astype(jnp.int32) + 25

    c = lax.broadcasted_iota(jnp.int32, (_K, l, bb), 0)
    ones = (c == t1) | (c == t2) | (c == t3) | (c == 49)
    feats = ones.astype(jnp.float32)
    feats += jnp.where(c == 46, xt[3:4], 0.0)
    feats += jnp.where(c == 47, xt[4:5], 0.0)
    feats += jnp.where(c == 48, xt[5:6], 0.0)

    y = lax.dot_general(feats, w_ref[...], (((0,), (0,)), ((), ())),
                        preferred_element_type=jnp.float32)
    out_ref[...] = jnp.maximum(y, w_ref[50:51, :][None])


def _pack_weights(amino_table, element_table, position_table, W_xyz, b_xyz):
    w = jnp.zeros((_K, _DOUT), dtype=jnp.float32)
    w = w.at[0:20, 0:_D].set(amino_table)
    w = w.at[20:25, _D:2 * _D].set(element_table)
    w = w.at[25:46, 2 * _D:3 * _D].set(position_table)
    w = w.at[46:49, 3 * _D:].set(W_xyz)
    w = w.at[49, 3 * _D:].set(b_xyz)
    w = w.at[50, 0:3 * _D].set(jnp.finfo(jnp.float32).min)
    return w


def kernel(x, amino_table, element_table, position_table, W_xyz, b_xyz):
    B, L, _ = x.shape
    w = _pack_weights(amino_table, element_table, position_table, W_xyz,
                      b_xyz)
    xt = jnp.transpose(x, (2, 1, 0))
    out_t = pl.pallas_call(
        _fused_body,
        grid=(B // _BB,),
        in_specs=[
            pl.BlockSpec((6, L, _BB), lambda i: (0, 0, i)),
            pl.BlockSpec((_K, _DOUT), lambda i: (0, 0)),
        ],
        out_specs=pl.BlockSpec((L, _BB, _DOUT), lambda i: (0, i, 0)),
        out_shape=jax.ShapeDtypeStruct((L, B, _DOUT), jnp.float32),
        compiler_params=pltpu.CompilerParams(
            dimension_semantics=("parallel",),
        ),
    )(xt, w)
    return jnp.transpose(out_t, (1, 0, 2))
